# baseline (speedup 1.0000x reference)
import dataclasses
import functools

import jax
import jax.numpy as jnp
from jax import lax
from jax.experimental import pallas as pl
from jax.experimental.pallas import tpu as pltpu
from jax.experimental.pallas import tpu_sc as plsc

B = 1024
N = 100000
L = 16
NC, NS = 2, 16
NW = NC * NS
BPW = B // NW

TN = 3584
GRID = -(-N // TN)


def _sc_gather_y(indexes, labels128):
    mesh = plsc.VectorSubcoreMesh(core_axis_name="c", subcore_axis_name="s")
    cp = pltpu.CompilerParams()
    if "needs_layout_passes" in pltpu.CompilerParams.__dataclass_fields__:
        cp = dataclasses.replace(cp, needs_layout_passes=False)

    @functools.partial(
        pl.kernel,
        out_type=jax.ShapeDtypeStruct((B,), jnp.int32),
        mesh=mesh,
        compiler_params=cp,
        scratch_types=[
            pltpu.VMEM((BPW,), jnp.int32),
            pltpu.VMEM((BPW,), jnp.int32),
            pltpu.VMEM((BPW, 128), jnp.int32),
            pltpu.VMEM((BPW,), jnp.int32),
            pltpu.SemaphoreType.DMA,
        ],
    )
    def k(idx_hbm, lab_hbm, y_hbm, idx_v, row_v, rows_v, y_v, sem):
        wid = lax.axis_index("s") * NC + lax.axis_index("c")
        base = wid * BPW
        pltpu.sync_copy(idx_hbm.at[pl.ds(base, BPW)], idx_v)
        for j in range(BPW // L):
            idxr = idx_v[pl.ds(j * L, L)]
            row_v[pl.ds(j * L, L)] = jax.lax.shift_right_logical(idxr, 7)
        pltpu.async_copy(lab_hbm.at[row_v], rows_v, sem).wait()
        for j in range(BPW // L):
            idxr = idx_v[pl.ds(j * L, L)]
            lane = jax.lax.bitwise_and(idxr, 127)
            rowsel = jax.lax.iota(jnp.int32, L) + j * L
            y_v[pl.ds(j * L, L)] = plsc.load_gather(rows_v, [rowsel, lane])
        pltpu.sync_copy(y_v, y_hbm.at[pl.ds(base, BPW)])

    return k(indexes, labels128)


def _sweep_body(idx_ref, y_ref, lab_ref, x_ref, out_ref, p_acc, z_acc):
    i = pl.program_id(0)

    @pl.when(i == 0)
    def _init():
        p_acc[...] = jnp.zeros_like(p_acc)
        z_acc[...] = jnp.zeros_like(z_acc)

    def fold128(t):
        s = t[:, 0:128]
        for k in range(1, TN // 128):
            s = s + t[:, k * 128:(k + 1) * 128]
        return s

    def accumulate(mask_pad):
        xe = jnp.exp(x_ref[...])
        col = lax.broadcasted_iota(jnp.int32, (1, TN), 1) + i * TN
        keep = col != idx_ref[...]
        if mask_pad:
            keep = keep & (col < N)
        e = jnp.where(keep, xe, 0.0)
        pe = jnp.where(lab_ref[...] == y_ref[...], e, 0.0)
        p_acc[...] += fold128(pe)
        z_acc[...] += fold128(e)

    @pl.when(i < GRID - 1)
    def _interior():
        accumulate(False)

    @pl.when(i == GRID - 1)
    def _last():
        accumulate(True)

    @pl.when(i == GRID - 1)
    def _fin():
        p = jnp.sum(p_acc[...], axis=1, keepdims=True)
        z = jnp.sum(z_acc[...], axis=1, keepdims=True)
        prob = p / z
        ok = prob != 0.0
        ll = jnp.where(ok, jnp.log(jnp.where(ok, prob, 1.0)), 0.0)
        out_ref[...] = -jnp.sum(ll, axis=0, keepdims=True) / B


def _tc_loss(x, indexes, labels, y):
    grid_spec = pltpu.PrefetchScalarGridSpec(
        num_scalar_prefetch=0,
        grid=(GRID,),
        in_specs=[
            pl.BlockSpec((B, 1), lambda i: (0, 0)),
            pl.BlockSpec((B, 1), lambda i: (0, 0)),
            pl.BlockSpec((1, TN), lambda i: (0, i)),
            pl.BlockSpec((B, TN), lambda i: (0, i)),
        ],
        out_specs=pl.BlockSpec((1, 1), lambda i: (0, 0)),
        scratch_shapes=[
            pltpu.VMEM((B, 128), jnp.float32),
            pltpu.VMEM((B, 128), jnp.float32),
        ],
    )
    out = pl.pallas_call(
        _sweep_body,
        grid_spec=grid_spec,
        out_shape=jax.ShapeDtypeStruct((1, 1), jnp.float32),
    )(indexes.reshape(B, 1), y.reshape(B, 1), labels.reshape(1, N), x)
    return out[0, 0]


def kernel(x, indexes, labels):
    npad = -N % 128
    labels128 = jnp.pad(labels, (0, npad)).reshape(-1, 128)
    y = _sc_gather_y(indexes, labels128)
    return _tc_loss(x, indexes, labels, y)


# speedup vs baseline: 1.1367x; 1.1367x over previous
import dataclasses
import functools

import jax
import jax.numpy as jnp
from jax import lax
from jax.experimental import pallas as pl
from jax.experimental.pallas import tpu as pltpu
from jax.experimental.pallas import tpu_sc as plsc

B = 1024
N = 100000
L = 16
NC, NS = 2, 16
NW = NC * NS
BPW = B // NW

TN = 2560
GRID = -(-N // TN)


def _sc_gather_y(indexes, labels128):
    mesh = plsc.VectorSubcoreMesh(core_axis_name="c", subcore_axis_name="s")
    cp = pltpu.CompilerParams()
    if "needs_layout_passes" in pltpu.CompilerParams.__dataclass_fields__:
        cp = dataclasses.replace(cp, needs_layout_passes=False)

    @functools.partial(
        pl.kernel,
        out_type=jax.ShapeDtypeStruct((B,), jnp.int32),
        mesh=mesh,
        compiler_params=cp,
        scratch_types=[
            pltpu.VMEM((BPW,), jnp.int32),
            pltpu.VMEM((BPW,), jnp.int32),
            pltpu.VMEM((BPW, 128), jnp.int32),
            pltpu.VMEM((BPW,), jnp.int32),
            pltpu.SemaphoreType.DMA,
        ],
    )
    def k(idx_hbm, lab_hbm, y_hbm, idx_v, row_v, rows_v, y_v, sem):
        wid = lax.axis_index("s") * NC + lax.axis_index("c")
        base = wid * BPW
        pltpu.sync_copy(idx_hbm.at[pl.ds(base, BPW)], idx_v)
        for j in range(BPW // L):
            idxr = idx_v[pl.ds(j * L, L)]
            row_v[pl.ds(j * L, L)] = jax.lax.shift_right_logical(idxr, 7)
        pltpu.async_copy(lab_hbm.at[row_v], rows_v, sem).wait()
        for j in range(BPW // L):
            idxr = idx_v[pl.ds(j * L, L)]
            lane = jax.lax.bitwise_and(idxr, 127)
            rowsel = jax.lax.iota(jnp.int32, L) + j * L
            y_v[pl.ds(j * L, L)] = plsc.load_gather(rows_v, [rowsel, lane])
        pltpu.sync_copy(y_v, y_hbm.at[pl.ds(base, BPW)])

    return k(indexes, labels128)


def _sweep_body(idx_ref, y_ref, lab_ref, x_ref, out_ref, p_acc, z_acc):
    i = pl.program_id(0)

    @pl.when(i == 0)
    def _init():
        p_acc[...] = jnp.zeros_like(p_acc)
        z_acc[...] = jnp.zeros_like(z_acc)

    def fold128(t):
        s = t[:, 0:128]
        for k in range(1, TN // 128):
            s = s + t[:, k * 128:(k + 1) * 128]
        return s

    def accumulate(mask_pad):
        xe = jnp.exp(x_ref[...])
        col = lax.broadcasted_iota(jnp.int32, (1, TN), 1) + i * TN
        keep = col != idx_ref[...]
        if mask_pad:
            keep = keep & (col < N)
        e = jnp.where(keep, xe, 0.0)
        pe = jnp.where(lab_ref[...] == y_ref[...], e, 0.0)
        p_acc[...] += fold128(pe)
        z_acc[...] += fold128(e)

    @pl.when(i < GRID - 1)
    def _interior():
        accumulate(False)

    @pl.when(i == GRID - 1)
    def _last():
        accumulate(True)

    @pl.when(i == GRID - 1)
    def _fin():
        p = jnp.sum(p_acc[...], axis=1, keepdims=True)
        z = jnp.sum(z_acc[...], axis=1, keepdims=True)
        prob = p / z
        ok = prob != 0.0
        ll = jnp.where(ok, jnp.log(jnp.where(ok, prob, 1.0)), 0.0)
        out_ref[...] = -jnp.sum(ll, axis=0, keepdims=True) / B


def _tc_loss(x, indexes, labels, y):
    grid_spec = pltpu.PrefetchScalarGridSpec(
        num_scalar_prefetch=0,
        grid=(GRID,),
        in_specs=[
            pl.BlockSpec((B, 1), lambda i: (0, 0)),
            pl.BlockSpec((B, 1), lambda i: (0, 0)),
            pl.BlockSpec((1, TN), lambda i: (0, i)),
            pl.BlockSpec((B, TN), lambda i: (0, i)),
        ],
        out_specs=pl.BlockSpec((1, 1), lambda i: (0, 0)),
        scratch_shapes=[
            pltpu.VMEM((B, 128), jnp.float32),
            pltpu.VMEM((B, 128), jnp.float32),
        ],
    )
    out = pl.pallas_call(
        _sweep_body,
        grid_spec=grid_spec,
        out_shape=jax.ShapeDtypeStruct((1, 1), jnp.float32),
    )(indexes.reshape(B, 1), y.reshape(B, 1), labels.reshape(1, N), x)
    return out[0, 0]


def kernel(x, indexes, labels):
    npad = -N % 128
    labels128 = jnp.pad(labels, (0, npad)).reshape(-1, 128)
    y = _sc_gather_y(indexes, labels128)
    return _tc_loss(x, indexes, labels, y)


# speedup vs baseline: 1.1483x; 1.0102x over previous
import dataclasses
import functools

import jax
import jax.numpy as jnp
from jax import lax
from jax.experimental import pallas as pl
from jax.experimental.pallas import tpu as pltpu
from jax.experimental.pallas import tpu_sc as plsc

B = 1024
N = 100000
L = 16
NC, NS = 2, 16
NW = NC * NS
BPW = B // NW

TN = 3072
GRID = -(-N // TN)


def _sc_gather_y(indexes, labels128):
    mesh = plsc.VectorSubcoreMesh(core_axis_name="c", subcore_axis_name="s")
    cp = pltpu.CompilerParams()
    if "needs_layout_passes" in pltpu.CompilerParams.__dataclass_fields__:
        cp = dataclasses.replace(cp, needs_layout_passes=False)

    @functools.partial(
        pl.kernel,
        out_type=jax.ShapeDtypeStruct((B,), jnp.int32),
        mesh=mesh,
        compiler_params=cp,
        scratch_types=[
            pltpu.VMEM((BPW,), jnp.int32),
            pltpu.VMEM((BPW,), jnp.int32),
            pltpu.VMEM((BPW, 128), jnp.int32),
            pltpu.VMEM((BPW,), jnp.int32),
            pltpu.SemaphoreType.DMA,
        ],
    )
    def k(idx_hbm, lab_hbm, y_hbm, idx_v, row_v, rows_v, y_v, sem):
        wid = lax.axis_index("s") * NC + lax.axis_index("c")
        base = wid * BPW
        pltpu.sync_copy(idx_hbm.at[pl.ds(base, BPW)], idx_v)
        for j in range(BPW // L):
            idxr = idx_v[pl.ds(j * L, L)]
            row_v[pl.ds(j * L, L)] = jax.lax.shift_right_logical(idxr, 7)
        pltpu.async_copy(lab_hbm.at[row_v], rows_v, sem).wait()
        for j in range(BPW // L):
            idxr = idx_v[pl.ds(j * L, L)]
            lane = jax.lax.bitwise_and(idxr, 127)
            rowsel = jax.lax.iota(jnp.int32, L) + j * L
            y_v[pl.ds(j * L, L)] = plsc.load_gather(rows_v, [rowsel, lane])
        pltpu.sync_copy(y_v, y_hbm.at[pl.ds(base, BPW)])

    return k(indexes, labels128)


def _sweep_body(idx_ref, y_ref, lab_ref, x_ref, out_ref, p_acc, z_acc):
    i = pl.program_id(0)

    @pl.when(i == 0)
    def _init():
        p_acc[...] = jnp.zeros_like(p_acc)
        z_acc[...] = jnp.zeros_like(z_acc)

    def fold128(t):
        s = t[:, 0:128]
        for k in range(1, TN // 128):
            s = s + t[:, k * 128:(k + 1) * 128]
        return s

    def accumulate(mask_pad):
        xe = jnp.exp(x_ref[...])
        col = lax.broadcasted_iota(jnp.int32, (1, TN), 1) + i * TN
        keep = col != idx_ref[...]
        if mask_pad:
            keep = keep & (col < N)
        e = jnp.where(keep, xe, 0.0)
        pe = jnp.where(lab_ref[...] == y_ref[...], e, 0.0)
        p_acc[...] += fold128(pe)
        z_acc[...] += fold128(e)

    @pl.when(i < GRID - 1)
    def _interior():
        accumulate(False)

    @pl.when(i == GRID - 1)
    def _last():
        accumulate(True)

    @pl.when(i == GRID - 1)
    def _fin():
        p = jnp.sum(p_acc[...], axis=1, keepdims=True)
        z = jnp.sum(z_acc[...], axis=1, keepdims=True)
        prob = p / z
        ok = prob != 0.0
        ll = jnp.where(ok, jnp.log(jnp.where(ok, prob, 1.0)), 0.0)
        out_ref[...] = -jnp.sum(ll, axis=0, keepdims=True) / B


def _tc_loss(x, indexes, labels, y):
    grid_spec = pltpu.PrefetchScalarGridSpec(
        num_scalar_prefetch=0,
        grid=(GRID,),
        in_specs=[
            pl.BlockSpec((B, 1), lambda i: (0, 0)),
            pl.BlockSpec((B, 1), lambda i: (0, 0)),
            pl.BlockSpec((1, TN), lambda i: (0, i)),
            pl.BlockSpec((B, TN), lambda i: (0, i)),
        ],
        out_specs=pl.BlockSpec((1, 1), lambda i: (0, 0)),
        scratch_shapes=[
            pltpu.VMEM((B, 128), jnp.float32),
            pltpu.VMEM((B, 128), jnp.float32),
        ],
    )
    out = pl.pallas_call(
        _sweep_body,
        grid_spec=grid_spec,
        out_shape=jax.ShapeDtypeStruct((1, 1), jnp.float32),
    )(indexes.reshape(B, 1), y.reshape(B, 1), labels.reshape(1, N), x)
    return out[0, 0]


def kernel(x, indexes, labels):
    npad = -N % 128
    labels128 = jnp.pad(labels, (0, npad)).reshape(-1, 128)
    y = _sc_gather_y(indexes, labels128)
    return _tc_loss(x, indexes, labels, y)
